# Initial kernel scaffold; baseline (speedup 1.0000x reference)
import jax
import jax.numpy as jnp
from jax.experimental import pallas as pl


def kernel(x, edge_index, edge_features, W, b):
    raise NotImplementedError("write your pallas kernel here")


# speedup vs baseline: 4.9172x; 4.9172x over previous
import functools

import jax
import jax.numpy as jnp
from jax import lax
from jax.experimental import pallas as pl
from jax.experimental.pallas import tpu as pltpu
from jax.experimental.pallas import tpu_sc as plsc

N_NODES = 10000
N_EDGES = 320000
D_FEAT = 128
D_HALF = D_FEAT // 2
D_EDGE = 16
OUT_DIM = 128

NC = 2
NS = 16
CHUNK = 80
CHUNKS_PER_TILE = N_EDGES // (NS * CHUNK)
N_PAD = 10240
ROWS_PER_TILE = N_PAD // NS

_sc_mesh = plsc.VectorSubcoreMesh(
    core_axis_name="c", subcore_axis_name="s", num_cores=NC)


@functools.partial(
    pl.kernel,
    out_type=(
        jax.ShapeDtypeStruct((NC, N_PAD, D_HALF), jnp.float32),
        jax.ShapeDtypeStruct((N_PAD, D_EDGE), jnp.float32),
        jax.ShapeDtypeStruct((N_PAD, 16), jnp.float32),
    ),
    mesh=_sc_mesh,
    compiler_params=pltpu.CompilerParams(use_tc_tiling_on_sc=False),
    scratch_types=(
        pltpu.VMEM((CHUNKS_PER_TILE, CHUNK), jnp.int32),
        pltpu.VMEM((CHUNKS_PER_TILE, CHUNK), jnp.int32),
        pltpu.VMEM((CHUNK, D_HALF), jnp.float32),
        pltpu.VMEM((CHUNK, D_EDGE), jnp.float32),
        pltpu.VMEM((CHUNK, 16), jnp.float32),
        pltpu.VMEM_SHARED((N_PAD, D_HALF), jnp.float32),
        pltpu.VMEM_SHARED((N_PAD, D_EDGE), jnp.float32),
        pltpu.VMEM_SHARED((N_PAD, 16), jnp.float32),
        pltpu.SemaphoreType.DMA,
        pltpu.SemaphoreType.DMA,
    ),
)
def _sc_accumulate(src_hbm, dst_hbm, x_hbm, ef_hbm, z64_hbm, z16_hbm, ones_hbm,
                   accx_hbm, acce_hbm, accc_hbm,
                   src_v, dst_v, xbuf, efbuf, onesbuf, shx, she, shc,
                   gsem, esem):
    c = lax.axis_index("c")
    s = lax.axis_index("s")

    r0 = s * ROWS_PER_TILE
    pltpu.sync_copy(z64_hbm, shx.at[pl.ds(r0, ROWS_PER_TILE)])
    @pl.when(c == 0)
    def _():
        pltpu.sync_copy(z16_hbm, she.at[pl.ds(r0, ROWS_PER_TILE)])

    @pl.when(c == 1)
    def _():
        pltpu.sync_copy(z16_hbm, shc.at[pl.ds(r0, ROWS_PER_TILE)])
        pltpu.sync_copy(ones_hbm, onesbuf)

    pltpu.sync_copy(src_hbm.at[s], src_v)
    pltpu.sync_copy(dst_hbm.at[s], dst_v)
    plsc.subcore_barrier()

    ef_base = s * CHUNKS_PER_TILE * CHUNK

    def chunk_body(j, carry):
        gcp = pltpu.async_copy(x_hbm.at[c].at[src_v.at[j]], xbuf, gsem)

        @pl.when(c == 0)
        def _():
            pltpu.sync_copy(
                ef_hbm.at[pl.ds(ef_base + j * CHUNK, CHUNK)], efbuf)

        gcp.wait()
        pltpu.sync_copy(xbuf, shx.at[dst_v.at[j]], add=True)

        @pl.when(c == 0)
        def _():
            pltpu.sync_copy(efbuf, she.at[dst_v.at[j]], add=True)

        @pl.when(c == 1)
        def _():
            pltpu.sync_copy(onesbuf, shc.at[dst_v.at[j]], add=True)

        return carry

    lax.fori_loop(0, CHUNKS_PER_TILE, chunk_body, 0)

    plsc.subcore_barrier()

    pltpu.sync_copy(shx.at[pl.ds(r0, ROWS_PER_TILE)],
                    accx_hbm.at[c].at[pl.ds(r0, ROWS_PER_TILE)])
    @pl.when(c == 0)
    def _():
        pltpu.sync_copy(she.at[pl.ds(r0, ROWS_PER_TILE)],
                        acce_hbm.at[pl.ds(r0, ROWS_PER_TILE)])

    @pl.when(c == 1)
    def _():
        pltpu.sync_copy(shc.at[pl.ds(r0, ROWS_PER_TILE)],
                        accc_hbm.at[pl.ds(r0, ROWS_PER_TILE)])


def _finish_body(accx_ref, acce_ref, accc_ref, wx_ref, we_ref, b_ref, out_ref):
    acc0 = accx_ref[0, :N_NODES]
    acc1 = accx_ref[1, :N_NODES]
    acce = acce_ref[:N_NODES]
    counts = accc_ref[:N_NODES, 0:1]
    sums = jnp.dot(acc0, wx_ref[:D_HALF], preferred_element_type=jnp.float32)
    sums = sums + jnp.dot(acc1, wx_ref[D_HALF:], preferred_element_type=jnp.float32)
    sums = sums + jnp.dot(acce, we_ref[...], preferred_element_type=jnp.float32)
    sums = sums + counts * b_ref[...]
    out_ref[...] = sums / jnp.maximum(counts, 1.0)


_finish = pl.pallas_call(
    _finish_body,
    out_shape=jax.ShapeDtypeStruct((N_NODES, OUT_DIM), jnp.float32),
)


def kernel(x, edge_index, edge_features, W, b):
    src = edge_index[0].astype(jnp.int32).reshape(NS, CHUNKS_PER_TILE, CHUNK)
    dst = edge_index[1].astype(jnp.int32).reshape(NS, CHUNKS_PER_TILE, CHUNK)
    xsplit = x.reshape(N_NODES, NC, D_HALF).transpose(1, 0, 2)
    z64 = jnp.zeros((ROWS_PER_TILE, D_HALF), jnp.float32)
    z16 = jnp.zeros((ROWS_PER_TILE, 16), jnp.float32)
    ones = jnp.ones((CHUNK, 16), jnp.float32)
    accx, acce, accc = _sc_accumulate(src, dst, xsplit, edge_features,
                                      z64, z16, ones)
    wx = W[:D_FEAT]
    we = W[D_FEAT:]
    return _finish(accx, acce, accc, wx, we, b.reshape(1, OUT_DIM))


# speedup vs baseline: 7.4941x; 1.5241x over previous
import functools

import jax
import jax.numpy as jnp
from jax import lax
from jax.experimental import pallas as pl
from jax.experimental.pallas import tpu as pltpu
from jax.experimental.pallas import tpu_sc as plsc

N_NODES = 10000
N_EDGES = 320000
D_FEAT = 128
D_HALF = D_FEAT // 2
D_EDGE = 16
OUT_DIM = 128

NC = 2
NS = 16
CHUNK = 125
EDGES_PER_TILE = N_EDGES // NS
NCH = EDGES_PER_TILE // CHUNK
N_PAD = 10240
ROWS_PER_TILE = N_PAD // NS

_sc_mesh = plsc.VectorSubcoreMesh(
    core_axis_name="c", subcore_axis_name="s", num_cores=NC)


@functools.partial(
    pl.kernel,
    out_type=(
        jax.ShapeDtypeStruct((NC, N_PAD, D_HALF), jnp.float32),
        jax.ShapeDtypeStruct((N_PAD, D_EDGE), jnp.float32),
        jax.ShapeDtypeStruct((N_PAD, 16), jnp.float32),
    ),
    mesh=_sc_mesh,
    compiler_params=pltpu.CompilerParams(use_tc_tiling_on_sc=False),
    scratch_types=(
        pltpu.VMEM((NCH, CHUNK), jnp.int32),
        pltpu.VMEM((NCH, CHUNK), jnp.int32),
        pltpu.VMEM((CHUNK, D_HALF), jnp.float32),
        pltpu.VMEM((CHUNK, D_HALF), jnp.float32),
        pltpu.VMEM((CHUNK, D_EDGE), jnp.float32),
        pltpu.VMEM((CHUNK, D_EDGE), jnp.float32),
        pltpu.VMEM((CHUNK, 16), jnp.float32),
        pltpu.VMEM_SHARED((N_PAD, D_HALF), jnp.float32),
        pltpu.VMEM_SHARED((N_PAD, D_EDGE), jnp.float32),
        pltpu.VMEM_SHARED((N_PAD, 16), jnp.float32),
        pltpu.SemaphoreType.DMA,
        pltpu.SemaphoreType.DMA,
        pltpu.SemaphoreType.DMA,
        pltpu.SemaphoreType.DMA,
    ),
)
def _sc_accumulate(src_hbm, dst_hbm, x_hbm, ef_hbm, z64_hbm, z16_hbm, ones_hbm,
                   accx_hbm, acce_hbm, accc_hbm,
                   src_v, dst_v, xbuf0, xbuf1, efbuf0, efbuf1, onesbuf,
                   shx, she, shc, gsem0, gsem1, esem0, esem1):
    c = lax.axis_index("c")
    s = lax.axis_index("s")

    r0 = s * ROWS_PER_TILE
    pltpu.sync_copy(z64_hbm, shx.at[pl.ds(r0, ROWS_PER_TILE)])
    @pl.when(c == 0)
    def _():
        pltpu.sync_copy(z16_hbm, she.at[pl.ds(r0, ROWS_PER_TILE)])

    @pl.when(c == 1)
    def _():
        pltpu.sync_copy(z16_hbm, shc.at[pl.ds(r0, ROWS_PER_TILE)])
        pltpu.sync_copy(ones_hbm, onesbuf)

    pltpu.sync_copy(src_hbm.at[s], src_v)
    pltpu.sync_copy(dst_hbm.at[s], dst_v)
    plsc.subcore_barrier()

    ef_base = s * EDGES_PER_TILE

    pltpu.async_copy(x_hbm.at[c].at[src_v.at[0]], xbuf0, gsem0)
    pltpu.async_copy(x_hbm.at[c].at[src_v.at[1]], xbuf1, gsem1)

    @pl.when(c == 0)
    def _():
        pltpu.async_copy(ef_hbm.at[pl.ds(ef_base, CHUNK)], efbuf0, esem0)
        pltpu.async_copy(ef_hbm.at[pl.ds(ef_base + CHUNK, CHUNK)], efbuf1, esem1)

    def process(ci, xbuf, efbuf, gsem, esem):
        pltpu.make_async_copy(x_hbm.at[c].at[src_v.at[0]], xbuf, gsem).wait()
        pltpu.sync_copy(xbuf, shx.at[dst_v.at[ci]], add=True)

        @pl.when(c == 0)
        def _():
            pltpu.make_async_copy(
                ef_hbm.at[pl.ds(ef_base, CHUNK)], efbuf, esem).wait()
            pltpu.sync_copy(efbuf, she.at[dst_v.at[ci]], add=True)

        @pl.when(c == 1)
        def _():
            pltpu.sync_copy(onesbuf, shc.at[dst_v.at[ci]], add=True)

        nxt = jnp.minimum(ci + 2, NCH - 1)
        pltpu.async_copy(x_hbm.at[c].at[src_v.at[nxt]], xbuf, gsem)

        @pl.when(c == 0)
        def _():
            pltpu.async_copy(
                ef_hbm.at[pl.ds(ef_base + nxt * CHUNK, CHUNK)], efbuf, esem)

    def pair_body(j, carry):
        a = 2 * j
        process(a, xbuf0, efbuf0, gsem0, esem0)
        process(a + 1, xbuf1, efbuf1, gsem1, esem1)
        return carry

    lax.fori_loop(0, NCH // 2, pair_body, 0)

    pltpu.make_async_copy(x_hbm.at[c].at[src_v.at[0]], xbuf0, gsem0).wait()
    pltpu.make_async_copy(x_hbm.at[c].at[src_v.at[0]], xbuf1, gsem1).wait()

    @pl.when(c == 0)
    def _():
        pltpu.make_async_copy(ef_hbm.at[pl.ds(ef_base, CHUNK)], efbuf0, esem0).wait()
        pltpu.make_async_copy(ef_hbm.at[pl.ds(ef_base, CHUNK)], efbuf1, esem1).wait()

    plsc.subcore_barrier()

    pltpu.sync_copy(shx.at[pl.ds(r0, ROWS_PER_TILE)],
                    accx_hbm.at[c].at[pl.ds(r0, ROWS_PER_TILE)])
    @pl.when(c == 0)
    def _():
        pltpu.sync_copy(she.at[pl.ds(r0, ROWS_PER_TILE)],
                        acce_hbm.at[pl.ds(r0, ROWS_PER_TILE)])

    @pl.when(c == 1)
    def _():
        pltpu.sync_copy(shc.at[pl.ds(r0, ROWS_PER_TILE)],
                        accc_hbm.at[pl.ds(r0, ROWS_PER_TILE)])


def _finish_body(accx_ref, acce_ref, accc_ref, wx_ref, we_ref, b_ref, out_ref):
    acc0 = accx_ref[0, :N_NODES]
    acc1 = accx_ref[1, :N_NODES]
    acce = acce_ref[:N_NODES]
    counts = accc_ref[:N_NODES, 0:1]
    sums = jnp.dot(acc0, wx_ref[:D_HALF], preferred_element_type=jnp.float32)
    sums = sums + jnp.dot(acc1, wx_ref[D_HALF:], preferred_element_type=jnp.float32)
    sums = sums + jnp.dot(acce, we_ref[...], preferred_element_type=jnp.float32)
    sums = sums + counts * b_ref[...]
    out_ref[...] = sums / jnp.maximum(counts, 1.0)


_finish = pl.pallas_call(
    _finish_body,
    out_shape=jax.ShapeDtypeStruct((N_NODES, OUT_DIM), jnp.float32),
)


def kernel(x, edge_index, edge_features, W, b):
    src = edge_index[0].astype(jnp.int32).reshape(NS, NCH, CHUNK)
    dst = edge_index[1].astype(jnp.int32).reshape(NS, NCH, CHUNK)
    xsplit = x.reshape(N_NODES, NC, D_HALF).transpose(1, 0, 2)
    z64 = jnp.zeros((ROWS_PER_TILE, D_HALF), jnp.float32)
    z16 = jnp.zeros((ROWS_PER_TILE, 16), jnp.float32)
    ones = jnp.ones((CHUNK, 16), jnp.float32)
    accx, acce, accc = _sc_accumulate(src, dst, xsplit, edge_features,
                                      z64, z16, ones)
    wx = W[:D_FEAT]
    we = W[D_FEAT:]
    return _finish(accx, acce, accc, wx, we, b.reshape(1, OUT_DIM))
